# baseline (speedup 1.0000x reference)
import jax
import jax.numpy as jnp
from jax.experimental import pallas as pl
from jax.experimental.pallas import tpu as pltpu

_B, _D, _T = 16, 256, 2048
_N_Q, _BINS = 8, 1024


_N_CHUNKS = 2


def _rvq_kernel(x_ref, cb_ref, q_ref, codes_ref, loss_ref):
    tc = _T // _N_CHUNKS
    nc = _N_CHUNKS
    rs = [x_ref[0, :, c * tc:(c + 1) * tc] for c in range(nc)]
    rns = [jnp.sum(r * r, axis=0) for r in rs]
    quants = [jnp.zeros_like(r) for r in rs]
    losses = []
    dn_s = (((1,), (0,)), ((), ()))
    dn_g = (((0,), (0,)), ((), ()))

    def gather_mm(cbstack, onehot):
        out = jax.lax.dot_general(cbstack, onehot, dn_g,
                                  preferred_element_type=jnp.float32)
        return (out[:_D] + out[_D:2 * _D]) + out[2 * _D:]

    for i in range(_N_Q):
        cb = cb_ref[i]
        cbn = jnp.sum(cb * cb, axis=1)
        a = cb.astype(jnp.bfloat16)
        r1 = cb - a.astype(jnp.float32)
        bb = r1.astype(jnp.bfloat16)
        cc = (r1 - bb.astype(jnp.float32)).astype(jnp.bfloat16)
        cbstack = jnp.concatenate([a, bb, cc], axis=1)

        ss = [jax.lax.dot_general(cb, rs[c], dn_s,
                                  preferred_element_type=jnp.float32)
              for c in range(nc)]
        idxs, qs = [None] * nc, [None] * nc
        for c in range(nc):
            d = (rns[c][None, :] - 2.0 * ss[c]) + cbn[:, None]
            idx = jnp.argmin(d, axis=0)
            onehot = (jax.lax.broadcasted_iota(jnp.int32, (_BINS, tc), 0)
                      == idx[None, :]).astype(jnp.bfloat16)
            idxs[c] = idx
            qs[c] = gather_mm(cbstack, onehot)
        layer_loss = None
        for c in range(nc):
            r, q = rs[c], qs[c]
            q_st = r + (q - r)
            cl = jnp.sum((q - r) ** 2)
            layer_loss = cl if layer_loss is None else layer_loss + cl
            quants[c] = quants[c] + q_st
            rs[c] = r - q_st
            rns[c] = jnp.sum(rs[c] * rs[c], axis=0)
            codes_ref[0, i, c * tc:(c + 1) * tc] = idxs[c]
        losses.append(layer_loss)
    for c in range(nc):
        q_ref[0, :, c * tc:(c + 1) * tc] = quants[c]
    loss_ref[0, 0, :] = jnp.stack(losses)


def _rvq_pallas(x, codebooks, nb):
    return pl.pallas_call(
        _rvq_kernel,
        grid=(nb,),
        in_specs=[
            pl.BlockSpec((1, _D, _T), lambda b: (b, 0, 0)),
            pl.BlockSpec((_N_Q, _BINS, _D), lambda b: (0, 0, 0)),
        ],
        out_specs=[
            pl.BlockSpec((1, _D, _T), lambda b: (b, 0, 0)),
            pl.BlockSpec((1, _N_Q, _T), lambda b: (b, 0, 0)),
            pl.BlockSpec((1, 1, _N_Q), lambda b: (b, 0, 0)),
        ],
        out_shape=[
            jax.ShapeDtypeStruct((nb, _D, _T), jnp.float32),
            jax.ShapeDtypeStruct((nb, _N_Q, _T), jnp.int32),
            jax.ShapeDtypeStruct((nb, 1, _N_Q), jnp.float32),
        ],
        compiler_params=pltpu.CompilerParams(
            dimension_semantics=("parallel",),
        ),
    )(x, codebooks)


def kernel(x, codebooks):
    devs = jax.devices()
    n_dev = 2 if len(devs) >= 2 and _B % 2 == 0 else 1
    mesh = jax.sharding.Mesh(devs[:n_dev], ("b",))
    P = jax.sharding.PartitionSpec

    def shard_fn(xs, cb):
        nb = _B // n_dev
        q_bdt, codes_bqt, loss_bq = _rvq_pallas(xs, cb, nb)
        codes = jnp.transpose(codes_bqt, (1, 0, 2))
        loss = jnp.sum(loss_bq[:, 0, :], axis=0, keepdims=True)
        return q_bdt, codes, loss

    q_bdt, codes, loss = jax.shard_map(
        shard_fn, mesh=mesh,
        in_specs=(P("b"), P()),
        out_specs=(P("b"), P(None, "b"), P("b")),
        check_vma=False,
    )(x, codebooks)
    commit_loss = jnp.mean(jnp.sum(loss, axis=0) / (_B * _T * _D))
    return q_bdt, codes, commit_loss


# speedup vs baseline: 1.0602x; 1.0602x over previous
import jax
import jax.numpy as jnp
from jax.experimental import pallas as pl
from jax.experimental.pallas import tpu as pltpu

_B, _D, _T = 16, 256, 2048
_N_Q, _BINS = 8, 1024


_N_CHUNKS = 2


def _rvq_kernel(x_ref, cb_ref, q_ref, codes_ref, loss_ref):
    tc = _T // _N_CHUNKS
    nc = _N_CHUNKS
    rs = [x_ref[0, :, c * tc:(c + 1) * tc] for c in range(nc)]
    rns = [jnp.sum(r * r, axis=0) for r in rs]
    quants = [jnp.zeros_like(r) for r in rs]
    losses = []
    dn_s = (((1,), (0,)), ((), ()))
    dn_g = (((0,), (0,)), ((), ()))

    def gather_mm(cbstack, onehot):
        out = jax.lax.dot_general(cbstack, onehot, dn_g,
                                  preferred_element_type=jnp.float32)
        return (out[:_D] + out[_D:2 * _D]) + out[2 * _D:]

    for i in range(_N_Q):
        cb = cb_ref[i]
        cbn = jnp.sum(cb * cb, axis=1)
        a = cb.astype(jnp.bfloat16)
        r1 = cb - a.astype(jnp.float32)
        bb = r1.astype(jnp.bfloat16)
        cc = (r1 - bb.astype(jnp.float32)).astype(jnp.bfloat16)
        cbstack = jnp.concatenate([a, bb, cc], axis=1)

        ss = [jax.lax.dot_general(cb, rs[c], dn_s,
                                  preferred_element_type=jnp.float32)
              for c in range(nc)]
        idxs, qs = [None] * nc, [None] * nc
        for c in range(nc):
            d = (rns[c][None, :] - 2.0 * ss[c]) + cbn[:, None]
            idx = jnp.argmin(d, axis=0)
            onehot = (jax.lax.broadcasted_iota(jnp.int32, (_BINS, tc), 0)
                      == idx[None, :]).astype(jnp.bfloat16)
            idxs[c] = idx
            qs[c] = gather_mm(cbstack, onehot)
        layer_loss = None
        for c in range(nc):
            r, q = rs[c], qs[c]
            q_st = r + (q - r)
            cl = jnp.sum((q - r) ** 2)
            layer_loss = cl if layer_loss is None else layer_loss + cl
            quants[c] = quants[c] + q_st
            rs[c] = r - q_st
            rns[c] = jnp.sum(rs[c] * rs[c], axis=0)
            codes_ref[0, i, c * tc:(c + 1) * tc] = idxs[c]
        losses.append(layer_loss)
    for c in range(nc):
        q_ref[0, :, c * tc:(c + 1) * tc] = quants[c]
    loss_ref[0, 0, :] = jnp.stack(losses)


def _rvq_pallas(x, codebooks, nb):
    return pl.pallas_call(
        _rvq_kernel,
        grid=(nb,),
        in_specs=[
            pl.BlockSpec((1, _D, _T), lambda b: (b, 0, 0)),
            pl.BlockSpec((_N_Q, _BINS, _D), lambda b: (0, 0, 0)),
        ],
        out_specs=[
            pl.BlockSpec((1, _D, _T), lambda b: (b, 0, 0)),
            pl.BlockSpec((1, _N_Q, _T), lambda b: (b, 0, 0)),
            pl.BlockSpec((1, 1, _N_Q), lambda b: (b, 0, 0)),
        ],
        out_shape=[
            jax.ShapeDtypeStruct((nb, _D, _T), jnp.float32),
            jax.ShapeDtypeStruct((nb, _N_Q, _T), jnp.int32),
            jax.ShapeDtypeStruct((nb, 1, _N_Q), jnp.float32),
        ],
        compiler_params=pltpu.CompilerParams(
            dimension_semantics=("parallel",),
        ),
    )(x, codebooks)


def kernel(x, codebooks):
    q_bdt, codes_bqt, loss_bq = _rvq_pallas(x, codebooks, _B)
    codes = jnp.transpose(codes_bqt, (1, 0, 2))
    commit_loss = jnp.mean(jnp.sum(loss_bq[:, 0, :], axis=0) / (_B * _T * _D))
    return q_bdt, codes, commit_loss


# speedup vs baseline: 1.1169x; 1.0535x over previous
import jax
import jax.numpy as jnp
from jax.experimental import pallas as pl
from jax.experimental.pallas import tpu as pltpu

_B, _D, _T = 16, 256, 2048
_N_Q, _BINS = 8, 1024


_N_CHUNKS = 4


def _rvq_kernel(x_ref, cb_ref, q_ref, codes_ref, loss_ref):
    tc = _T // _N_CHUNKS
    nc = _N_CHUNKS
    rs = [x_ref[0, :, c * tc:(c + 1) * tc] for c in range(nc)]
    rns = [jnp.sum(r * r, axis=0) for r in rs]
    quants = [jnp.zeros_like(r) for r in rs]
    losses = []
    dn_s = (((1,), (0,)), ((), ()))
    dn_g = (((0,), (0,)), ((), ()))

    def gather_mm(cbstack, onehot):
        out = jax.lax.dot_general(cbstack, onehot, dn_g,
                                  preferred_element_type=jnp.float32)
        return (out[:_D] + out[_D:2 * _D]) + out[2 * _D:]

    for i in range(_N_Q):
        cb = cb_ref[i]
        cbn = jnp.sum(cb * cb, axis=1)
        a = cb.astype(jnp.bfloat16)
        r1 = cb - a.astype(jnp.float32)
        bb = r1.astype(jnp.bfloat16)
        cc = (r1 - bb.astype(jnp.float32)).astype(jnp.bfloat16)
        cbstack = jnp.concatenate([a, bb, cc], axis=1)

        ss = [jax.lax.dot_general(cb, rs[c], dn_s,
                                  preferred_element_type=jnp.float32)
              for c in range(nc)]
        idxs, qs = [None] * nc, [None] * nc
        for c in range(nc):
            d = (rns[c][None, :] - 2.0 * ss[c]) + cbn[:, None]
            idx = jnp.argmin(d, axis=0)
            onehot = (jax.lax.broadcasted_iota(jnp.int32, (_BINS, tc), 0)
                      == idx[None, :]).astype(jnp.bfloat16)
            idxs[c] = idx
            qs[c] = gather_mm(cbstack, onehot)
        layer_loss = None
        for c in range(nc):
            r, q = rs[c], qs[c]
            q_st = r + (q - r)
            cl = jnp.sum((q - r) ** 2)
            layer_loss = cl if layer_loss is None else layer_loss + cl
            quants[c] = quants[c] + q_st
            rs[c] = r - q_st
            rns[c] = jnp.sum(rs[c] * rs[c], axis=0)
            codes_ref[0, i, c * tc:(c + 1) * tc] = idxs[c]
        losses.append(layer_loss)
    for c in range(nc):
        q_ref[0, :, c * tc:(c + 1) * tc] = quants[c]
    loss_ref[0, 0, :] = jnp.stack(losses)


def _rvq_pallas(x, codebooks, nb):
    return pl.pallas_call(
        _rvq_kernel,
        grid=(nb,),
        in_specs=[
            pl.BlockSpec((1, _D, _T), lambda b: (b, 0, 0)),
            pl.BlockSpec((_N_Q, _BINS, _D), lambda b: (0, 0, 0)),
        ],
        out_specs=[
            pl.BlockSpec((1, _D, _T), lambda b: (b, 0, 0)),
            pl.BlockSpec((1, _N_Q, _T), lambda b: (b, 0, 0)),
            pl.BlockSpec((1, 1, _N_Q), lambda b: (b, 0, 0)),
        ],
        out_shape=[
            jax.ShapeDtypeStruct((nb, _D, _T), jnp.float32),
            jax.ShapeDtypeStruct((nb, _N_Q, _T), jnp.int32),
            jax.ShapeDtypeStruct((nb, 1, _N_Q), jnp.float32),
        ],
        compiler_params=pltpu.CompilerParams(
            dimension_semantics=("parallel",),
        ),
    )(x, codebooks)


def kernel(x, codebooks):
    q_bdt, codes_bqt, loss_bq = _rvq_pallas(x, codebooks, _B)
    codes = jnp.transpose(codes_bqt, (1, 0, 2))
    commit_loss = jnp.mean(jnp.sum(loss_bq[:, 0, :], axis=0) / (_B * _T * _D))
    return q_bdt, codes, commit_loss
